# baseline (speedup 1.0000x reference)
import functools
import math

import jax
import jax.numpy as jnp
from jax import lax
from jax.experimental import pallas as pl
from jax.experimental.pallas import tpu as pltpu
from jax.experimental.pallas import tpu_sc as plsc

N = 10000
K = 32
E = N * K
D = 128
H = 4
DK = D // H

BN = 200
BE = BN * K
GRID = N // BN
SLICES = (2, 12, 12, 12, 12)
S = len(SLICES)
OFFS = tuple(sum(SLICES[:i]) for i in range(S))

_INV_SQRT_DK = 1.0 / math.sqrt(DK)


def _dot(a, b):
    return jnp.dot(a, b, preferred_element_type=jnp.float32)


def _dot_bf(a, b):
    return jnp.dot(a.astype(jnp.bfloat16), b.astype(jnp.bfloat16),
                   preferred_element_type=jnp.float32)


def _pairswap(x):
    r, c = x.shape
    up = jnp.roll(x, -1, axis=0)
    dn = jnp.roll(x, 1, axis=0)
    row = lax.broadcasted_iota(jnp.int32, (r, c), 0)
    return jnp.where(row % 2 == 0, up, dn)


def _head_blockdiag():
    d = lax.broadcasted_iota(jnp.int32, (D, D), 0)
    c = lax.broadcasted_iota(jnp.int32, (D, D), 1)
    return jnp.where(d // DK == c // K, 1.0, 0.0).astype(jnp.float32)


def _segsum_k(x):
    return x.reshape(x.shape[0] // K, K, D).sum(axis=1)


def _mha_residual(fh, mail, Wq, bq, Wk, bk, Wv, bv, Wo, bo):
    q = _dot(fh, Wq) + bq
    k = _dot_bf(mail, Wk) + bk
    v = _dot_bf(mail, Wv) + bv
    qe = jnp.broadcast_to(q[:, None, :], (BN, K, D)).reshape(BE, D)
    s = _dot(qe * k, _head_blockdiag()) * _INV_SQRT_DK
    u = jnp.exp(s)
    numer = _segsum_k(u * v)
    denom = _segsum_k(u)
    o = numer / denom
    return _dot(o, Wo) + bo + fh


def _p1_body(xe_ref, f_ref, Wq_ref, bq_ref, Wk_ref, bk_ref, Wv_ref, bv_ref,
             Wo_ref, bo_ref, fh1_ref, xeb_ref):
    xe = xe_ref[...]
    fh1_ref[...] = _mha_residual(
        f_ref[...], xe,
        Wq_ref[...], bq_ref[...], Wk_ref[...], bk_ref[...],
        Wv_ref[...], bv_ref[...], Wo_ref[...], bo_ref[...])
    xeb_ref[...] = xe.astype(jnp.bfloat16)


def _p2_body(xeb_ref, g_ref, fh1_ref, Wq_ref, bq_ref, Wk_ref, bk_ref,
             Wv_ref, bv_ref, Wo_ref, bo_ref, W0_ref, b0_ref,
             h1b_ref, fh2_ref):
    xe = xeb_ref[...].astype(jnp.float32)
    m = g_ref[...] - _pairswap(xe)
    h1 = jnp.maximum(xe + _dot(m, W0_ref[...]) + b0_ref[...], 0.0)
    h1b_ref[...] = h1.astype(jnp.bfloat16)
    fh2_ref[...] = _mha_residual(
        fh1_ref[...], h1,
        Wq_ref[...], bq_ref[...], Wk_ref[...], bk_ref[...],
        Wv_ref[...], bv_ref[...], Wo_ref[...], bo_ref[...])


def _p3_body(xeb_ref, g_ref, h1b_ref, fh2_ref, f_ref, W1_ref, b1_ref,
             Wl_ref, bl_ref, out_ref):
    xe = xeb_ref[...].astype(jnp.float32)
    h1 = h1b_ref[...].astype(jnp.float32)
    m = g_ref[...] - _pairswap(h1)
    h2 = jnp.maximum(xe + _dot(m, W1_ref[...]) + b1_ref[...], 0.0)
    mail_sum = _segsum_k(h2)
    Wl = Wl_ref[...]
    out_ref[...] = (_dot(mail_sum, Wl[0:D]) + _dot(fh2_ref[...], Wl[D:2 * D])
                    + _dot(f_ref[...], Wl[2 * D:3 * D]) + bl_ref[...])


def _edge_spec(off):
    return pl.BlockSpec((BE, D), lambda i, o=off: (i + o, 0))


def _node_spec(off):
    return pl.BlockSpec((BN, D), lambda i, o=off: (i + o, 0))


def _w_spec(rows):
    return pl.BlockSpec((rows, D), lambda i: (0, 0))


def _b_spec():
    return pl.BlockSpec((1, D), lambda i: (0, 0))


def _make_sc_gather(rows_total):
    info = plsc.get_sparse_core_info()
    nw = info.num_cores * info.num_subcores
    per_w = rows_total // nw
    ch = 200
    n_ch = per_w // ch
    pairs = n_ch // 2
    tail = n_ch - 2 * pairs
    mesh = plsc.VectorSubcoreMesh(core_axis_name="c", subcore_axis_name="s")

    @functools.partial(
        pl.kernel,
        out_type=jax.ShapeDtypeStruct((rows_total, D), jnp.float32),
        mesh=mesh,
        scratch_types=[
            pltpu.VMEM((ch,), jnp.int32),
            pltpu.VMEM((ch,), jnp.int32),
            pltpu.VMEM((ch, D), jnp.float32),
            pltpu.VMEM((ch, D), jnp.float32),
            pltpu.SemaphoreType.DMA,
            pltpu.SemaphoreType.DMA,
            pltpu.SemaphoreType.DMA,
            pltpu.SemaphoreType.DMA,
        ],
    )
    def gather(table_hbm, idx_hbm, out_hbm, idx_a, idx_b, rows_a, rows_b,
               gs_a, gs_b, ss_a, ss_b):
        wid = lax.axis_index("s") * info.num_cores + lax.axis_index("c")
        base = wid * per_w
        idx_v = (idx_a, idx_b)
        rows_v = (rows_a, rows_b)
        gs = (gs_a, gs_b)
        ss = (ss_a, ss_b)

        def store_wait(b):
            pltpu.make_async_copy(rows_v[b], out_hbm.at[pl.ds(base, ch)],
                                  ss[b]).wait()

        def body(i, _):
            @pl.when(i > 0)
            def _():
                for b in range(2):
                    store_wait(b)
            handles = []
            for b in range(2):
                off = base + (2 * i + b) * ch
                pltpu.sync_copy(idx_hbm.at[pl.ds(off, ch)], idx_v[b])
                handles.append(
                    pltpu.async_copy(table_hbm.at[idx_v[b]], rows_v[b],
                                     gs[b]))
            for b in range(2):
                off = base + (2 * i + b) * ch
                handles[b].wait()
                pltpu.async_copy(rows_v[b], out_hbm.at[pl.ds(off, ch)],
                                 ss[b])
            return ()

        lax.fori_loop(0, pairs, body, ())
        for b in range(2):
            store_wait(b)
        if tail:
            off = base + 2 * pairs * ch
            pltpu.sync_copy(idx_hbm.at[pl.ds(off, ch)], idx_a)
            pltpu.async_copy(table_hbm.at[idx_a], rows_a, gs_a).wait()
            pltpu.sync_copy(rows_a, out_hbm.at[pl.ds(off, ch)])

    return gather


def kernel(f, x_e, src, Wq, bq, Wk, bk, Wv, bv, Wo, bo, W0, b0, W1, b1,
           Wl, bl):
    bq2, bk2, bv2, bo2, b02, b12, bl2 = (
        b.reshape(1, D) for b in (bq, bk, bv, bo, b0, b1, bl))

    p1 = pl.pallas_call(
        _p1_body,
        grid=(GRID,),
        in_specs=[_edge_spec(0), _node_spec(0),
                  _w_spec(D), _b_spec(), _w_spec(D), _b_spec(),
                  _w_spec(D), _b_spec(), _w_spec(D), _b_spec()],
        out_specs=[pl.BlockSpec((BN, D), lambda i: (i, 0)),
                   pl.BlockSpec((BE, D), lambda i: (i, 0))],
        out_shape=[jax.ShapeDtypeStruct((N, D), jnp.float32),
                   jax.ShapeDtypeStruct((E, D), jnp.bfloat16)],
    )
    fh1, xeb = p1(x_e, f, Wq, bq2, Wk, bk2, Wv, bv2, Wo, bo2)

    gathers = {u: _make_sc_gather(u * BE) for u in set(SLICES)}
    src_s = [src[OFFS[s] * BE:(OFFS[s] + SLICES[s]) * BE] for s in range(S)]

    def xeb_spec(off):
        return pl.BlockSpec((BE, D), lambda i, o=off: (i + o, 0))

    def p2_slice(s, g):
        off = OFFS[s]
        u = SLICES[s]
        call = pl.pallas_call(
            _p2_body,
            grid=(u,),
            in_specs=[xeb_spec(off),
                      pl.BlockSpec((BE, D), lambda i: (i, 0)),
                      _node_spec(off),
                      _w_spec(D), _b_spec(), _w_spec(D), _b_spec(),
                      _w_spec(D), _b_spec(), _w_spec(D), _b_spec(),
                      _w_spec(D), _b_spec()],
            out_specs=[pl.BlockSpec((BE, D), lambda i: (i, 0)),
                       pl.BlockSpec((BN, D), lambda i: (i, 0))],
            out_shape=[jax.ShapeDtypeStruct((u * BE, D), jnp.bfloat16),
                       jax.ShapeDtypeStruct((u * BN, D), jnp.float32)],
        )
        return call(xeb, g, fh1, Wq, bq2, Wk, bk2, Wv, bv2, Wo, bo2,
                    W0, b02)

    def p3_slice(s, g, h1b, fh2s):
        off = OFFS[s]
        u = SLICES[s]
        call = pl.pallas_call(
            _p3_body,
            grid=(u,),
            in_specs=[xeb_spec(off),
                      pl.BlockSpec((BE, D), lambda i: (i, 0)),
                      pl.BlockSpec((BE, D), lambda i: (i, 0)),
                      pl.BlockSpec((BN, D), lambda i: (i, 0)),
                      _node_spec(off), _w_spec(D), _b_spec(),
                      pl.BlockSpec((3 * D, D), lambda i: (0, 0)), _b_spec()],
            out_specs=pl.BlockSpec((BN, D), lambda i: (i, 0)),
            out_shape=jax.ShapeDtypeStruct((u * BN, D), jnp.float32),
        )
        return call(xeb, g, h1b, fh2s, f, W1, b12, Wl, bl2)

    g0 = [gathers[SLICES[s]](fh1, src_s[s]) for s in range(S)]
    p2_out = [p2_slice(s, g0[s]) for s in range(S)]
    h1b = [o[0] for o in p2_out]
    fh2s = [o[1] for o in p2_out]
    fh2 = jnp.concatenate(fh2s, axis=0)

    g1 = [gathers[SLICES[s]](fh2, src_s[s]) for s in range(S)]
    outs = [p3_slice(s, g1[s], h1b[s], fh2s[s]) for s in range(S)]
    return jnp.concatenate(outs, axis=0)


# speedup vs baseline: 1.0174x; 1.0174x over previous
import functools
import math

import jax
import jax.numpy as jnp
from jax import lax
from jax.experimental import pallas as pl
from jax.experimental.pallas import tpu as pltpu
from jax.experimental.pallas import tpu_sc as plsc

N = 10000
K = 32
E = N * K
D = 128
H = 4
DK = D // H

BN = 200
BE = BN * K
GRID = N // BN
SLICES = (10, 10, 10, 10, 10)
S = len(SLICES)
OFFS = tuple(sum(SLICES[:i]) for i in range(S))

_INV_SQRT_DK = 1.0 / math.sqrt(DK)


def _dot(a, b):
    return jnp.dot(a, b, preferred_element_type=jnp.float32)


def _dot_bf(a, b):
    return jnp.dot(a.astype(jnp.bfloat16), b.astype(jnp.bfloat16),
                   preferred_element_type=jnp.float32)


def _pairswap(x):
    r, c = x.shape
    up = jnp.roll(x, -1, axis=0)
    dn = jnp.roll(x, 1, axis=0)
    row = lax.broadcasted_iota(jnp.int32, (r, c), 0)
    return jnp.where(row % 2 == 0, up, dn)


def _head_blockdiag():
    d = lax.broadcasted_iota(jnp.int32, (D, D), 0)
    c = lax.broadcasted_iota(jnp.int32, (D, D), 1)
    return jnp.where(d // DK == c // K, 1.0, 0.0).astype(jnp.float32)


def _segsum_k(x):
    return x.reshape(x.shape[0] // K, K, D).sum(axis=1)


def _mha_residual(fh, mail, Wq, bq, Wk, bk, Wv, bv, Wo, bo):
    q = _dot(fh, Wq) + bq
    k = _dot_bf(mail, Wk) + bk
    v = _dot_bf(mail, Wv) + bv
    qe = jnp.broadcast_to(q[:, None, :], (BN, K, D)).reshape(BE, D)
    s = _dot(qe * k, _head_blockdiag()) * _INV_SQRT_DK
    u = jnp.exp(s)
    numer = _segsum_k(u * v)
    denom = _segsum_k(u)
    o = numer / denom
    return _dot(o, Wo) + bo + fh


def _p1_body(xe_ref, f_ref, Wq_ref, bq_ref, Wk_ref, bk_ref, Wv_ref, bv_ref,
             Wo_ref, bo_ref, fh1_ref, xeb_ref):
    xe = xe_ref[...]
    fh1_ref[...] = _mha_residual(
        f_ref[...], xe,
        Wq_ref[...], bq_ref[...], Wk_ref[...], bk_ref[...],
        Wv_ref[...], bv_ref[...], Wo_ref[...], bo_ref[...])
    xeb_ref[...] = xe.astype(jnp.bfloat16)


def _p2_body(xeb_ref, g_ref, fh1_ref, Wq_ref, bq_ref, Wk_ref, bk_ref,
             Wv_ref, bv_ref, Wo_ref, bo_ref, W0_ref, b0_ref,
             h1b_ref, fh2_ref):
    xe = xeb_ref[...].astype(jnp.float32)
    m = g_ref[...] - _pairswap(xe)
    h1 = jnp.maximum(xe + _dot(m, W0_ref[...]) + b0_ref[...], 0.0)
    h1b_ref[...] = h1.astype(jnp.bfloat16)
    fh2_ref[...] = _mha_residual(
        fh1_ref[...], h1,
        Wq_ref[...], bq_ref[...], Wk_ref[...], bk_ref[...],
        Wv_ref[...], bv_ref[...], Wo_ref[...], bo_ref[...])


def _p3_body(xeb_ref, g_ref, h1b_ref, fh2_ref, f_ref, W1_ref, b1_ref,
             Wl_ref, bl_ref, out_ref):
    xe = xeb_ref[...].astype(jnp.float32)
    h1 = h1b_ref[...].astype(jnp.float32)
    m = g_ref[...] - _pairswap(h1)
    h2 = jnp.maximum(xe + _dot(m, W1_ref[...]) + b1_ref[...], 0.0)
    mail_sum = _segsum_k(h2)
    Wl = Wl_ref[...]
    out_ref[...] = (_dot(mail_sum, Wl[0:D]) + _dot(fh2_ref[...], Wl[D:2 * D])
                    + _dot(f_ref[...], Wl[2 * D:3 * D]) + bl_ref[...])


def _edge_spec(off):
    return pl.BlockSpec((BE, D), lambda i, o=off: (i + o, 0))


def _node_spec(off):
    return pl.BlockSpec((BN, D), lambda i, o=off: (i + o, 0))


def _w_spec(rows):
    return pl.BlockSpec((rows, D), lambda i: (0, 0))


def _b_spec():
    return pl.BlockSpec((1, D), lambda i: (0, 0))


def _make_sc_gather(rows_total):
    info = plsc.get_sparse_core_info()
    nw = info.num_cores * info.num_subcores
    per_w = rows_total // nw
    ch = 200
    n_ch = per_w // ch
    pairs = n_ch // 2
    tail = n_ch - 2 * pairs
    mesh = plsc.VectorSubcoreMesh(core_axis_name="c", subcore_axis_name="s")

    @functools.partial(
        pl.kernel,
        out_type=jax.ShapeDtypeStruct((rows_total, D), jnp.float32),
        mesh=mesh,
        scratch_types=[
            pltpu.VMEM((ch,), jnp.int32),
            pltpu.VMEM((ch,), jnp.int32),
            pltpu.VMEM((ch, D), jnp.float32),
            pltpu.VMEM((ch, D), jnp.float32),
            pltpu.SemaphoreType.DMA,
            pltpu.SemaphoreType.DMA,
            pltpu.SemaphoreType.DMA,
            pltpu.SemaphoreType.DMA,
        ],
    )
    def gather(table_hbm, idx_hbm, out_hbm, idx_a, idx_b, rows_a, rows_b,
               gs_a, gs_b, ss_a, ss_b):
        wid = lax.axis_index("s") * info.num_cores + lax.axis_index("c")
        base = wid * per_w
        idx_v = (idx_a, idx_b)
        rows_v = (rows_a, rows_b)
        gs = (gs_a, gs_b)
        ss = (ss_a, ss_b)

        def store_wait(b):
            pltpu.make_async_copy(rows_v[b], out_hbm.at[pl.ds(base, ch)],
                                  ss[b]).wait()

        def body(i, _):
            @pl.when(i > 0)
            def _():
                for b in range(2):
                    store_wait(b)
            handles = []
            for b in range(2):
                off = base + (2 * i + b) * ch
                pltpu.sync_copy(idx_hbm.at[pl.ds(off, ch)], idx_v[b])
                handles.append(
                    pltpu.async_copy(table_hbm.at[idx_v[b]], rows_v[b],
                                     gs[b]))
            for b in range(2):
                off = base + (2 * i + b) * ch
                handles[b].wait()
                pltpu.async_copy(rows_v[b], out_hbm.at[pl.ds(off, ch)],
                                 ss[b])
            return ()

        lax.fori_loop(0, pairs, body, ())
        for b in range(2):
            store_wait(b)
        if tail:
            off = base + 2 * pairs * ch
            pltpu.sync_copy(idx_hbm.at[pl.ds(off, ch)], idx_a)
            pltpu.async_copy(table_hbm.at[idx_a], rows_a, gs_a).wait()
            pltpu.sync_copy(rows_a, out_hbm.at[pl.ds(off, ch)])

    return gather


def kernel(f, x_e, src, Wq, bq, Wk, bk, Wv, bv, Wo, bo, W0, b0, W1, b1,
           Wl, bl):
    bq2, bk2, bv2, bo2, b02, b12, bl2 = (
        b.reshape(1, D) for b in (bq, bk, bv, bo, b0, b1, bl))

    p1 = pl.pallas_call(
        _p1_body,
        grid=(GRID,),
        in_specs=[_edge_spec(0), _node_spec(0),
                  _w_spec(D), _b_spec(), _w_spec(D), _b_spec(),
                  _w_spec(D), _b_spec(), _w_spec(D), _b_spec()],
        out_specs=[pl.BlockSpec((BN, D), lambda i: (i, 0)),
                   pl.BlockSpec((BE, D), lambda i: (i, 0))],
        out_shape=[jax.ShapeDtypeStruct((N, D), jnp.float32),
                   jax.ShapeDtypeStruct((E, D), jnp.bfloat16)],
    )
    fh1, xeb = p1(x_e, f, Wq, bq2, Wk, bk2, Wv, bv2, Wo, bo2)

    gathers = {u: _make_sc_gather(u * BE) for u in set(SLICES)}
    src_s = [src[OFFS[s] * BE:(OFFS[s] + SLICES[s]) * BE] for s in range(S)]

    def xeb_spec(off):
        return pl.BlockSpec((BE, D), lambda i, o=off: (i + o, 0))

    def p2_slice(s, g):
        off = OFFS[s]
        u = SLICES[s]
        call = pl.pallas_call(
            _p2_body,
            grid=(u,),
            in_specs=[xeb_spec(off),
                      pl.BlockSpec((BE, D), lambda i: (i, 0)),
                      _node_spec(off),
                      _w_spec(D), _b_spec(), _w_spec(D), _b_spec(),
                      _w_spec(D), _b_spec(), _w_spec(D), _b_spec(),
                      _w_spec(D), _b_spec()],
            out_specs=[pl.BlockSpec((BE, D), lambda i: (i, 0)),
                       pl.BlockSpec((BN, D), lambda i: (i, 0))],
            out_shape=[jax.ShapeDtypeStruct((u * BE, D), jnp.bfloat16),
                       jax.ShapeDtypeStruct((u * BN, D), jnp.float32)],
        )
        return call(xeb, g, fh1, Wq, bq2, Wk, bk2, Wv, bv2, Wo, bo2,
                    W0, b02)

    def p3_slice(s, g, h1b, fh2s):
        off = OFFS[s]
        u = SLICES[s]
        call = pl.pallas_call(
            _p3_body,
            grid=(u,),
            in_specs=[xeb_spec(off),
                      pl.BlockSpec((BE, D), lambda i: (i, 0)),
                      pl.BlockSpec((BE, D), lambda i: (i, 0)),
                      pl.BlockSpec((BN, D), lambda i: (i, 0)),
                      _node_spec(off), _w_spec(D), _b_spec(),
                      pl.BlockSpec((3 * D, D), lambda i: (0, 0)), _b_spec()],
            out_specs=pl.BlockSpec((BN, D), lambda i: (i, 0)),
            out_shape=jax.ShapeDtypeStruct((u * BN, D), jnp.float32),
        )
        return call(xeb, g, h1b, fh2s, f, W1, b12, Wl, bl2)

    g0 = [gathers[SLICES[s]](fh1, src_s[s]) for s in range(S)]
    p2_out = [p2_slice(s, g0[s]) for s in range(S)]
    h1b = [o[0] for o in p2_out]
    fh2s = [o[1] for o in p2_out]
    fh2 = jnp.concatenate(fh2s, axis=0)

    g1 = [gathers[SLICES[s]](fh2, src_s[s]) for s in range(S)]
    outs = [p3_slice(s, g1[s], h1b[s], fh2s[s]) for s in range(S)]
    return jnp.concatenate(outs, axis=0)


# speedup vs baseline: 1.0511x; 1.0331x over previous
import functools
import math

import jax
import jax.numpy as jnp
from jax import lax
from jax.experimental import pallas as pl
from jax.experimental.pallas import tpu as pltpu
from jax.experimental.pallas import tpu_sc as plsc

N = 10000
K = 32
E = N * K
D = 128
H = 4
DK = D // H

BN = 200
BE = BN * K
GRID = N // BN
SLICES = (10, 10, 10, 10, 10)
S = len(SLICES)
OFFS = tuple(sum(SLICES[:i]) for i in range(S))

_INV_SQRT_DK = 1.0 / math.sqrt(DK)


def _dot(a, b):
    return jnp.dot(a, b, preferred_element_type=jnp.float32)


def _dot_bf(a, b):
    return jnp.dot(a.astype(jnp.bfloat16), b.astype(jnp.bfloat16),
                   preferred_element_type=jnp.float32)


def _pairswap(x):
    r, c = x.shape
    up = jnp.roll(x, -1, axis=0)
    dn = jnp.roll(x, 1, axis=0)
    row = lax.broadcasted_iota(jnp.int32, (r, c), 0)
    return jnp.where(row % 2 == 0, up, dn)


def _head_blockdiag():
    d = lax.broadcasted_iota(jnp.int32, (D, D), 0)
    c = lax.broadcasted_iota(jnp.int32, (D, D), 1)
    return jnp.where(d // DK == c // K, 1.0, 0.0).astype(jnp.float32)


def _segsum_k(x):
    return x.reshape(x.shape[0] // K, K, D).sum(axis=1)


def _mha_residual(fh, mail, Wq, bq, Wk, bk, Wv, bv, Wo, bo):
    q = _dot(fh, Wq) + bq
    k = _dot(mail, Wk) + bk
    v = _dot(mail, Wv) + bv
    bn = fh.shape[0]
    qe = jnp.broadcast_to(q[:, None, :], (bn, K, D)).reshape(bn * K, D)
    s = _dot(qe * k, _head_blockdiag()) * _INV_SQRT_DK
    u = jnp.exp(s)
    numer = _segsum_k(u * v)
    denom = _segsum_k(u)
    o = numer / denom
    return _dot(o, Wo) + bo + fh


def _p1_body(xe_ref, f_ref, Wq_ref, bq_ref, Wk_ref, bk_ref, Wv_ref, bv_ref,
             Wo_ref, bo_ref, fh1_ref, xeb_ref):
    xe = xe_ref[...]
    fh1_ref[...] = _mha_residual(
        f_ref[...], xe,
        Wq_ref[...], bq_ref[...], Wk_ref[...], bk_ref[...],
        Wv_ref[...], bv_ref[...], Wo_ref[...], bo_ref[...])
    xeb_ref[...] = xe.astype(jnp.bfloat16)


def _p2_body(xeb_ref, g_ref, fh1_ref, Wq_ref, bq_ref, Wk_ref, bk_ref,
             Wv_ref, bv_ref, Wo_ref, bo_ref, W0_ref, b0_ref,
             h1b_ref, fh2_ref):
    xe = xeb_ref[...].astype(jnp.float32)
    m = g_ref[...] - _pairswap(xe)
    h1 = jnp.maximum(xe + _dot(m, W0_ref[...]) + b0_ref[...], 0.0)
    h1b_ref[...] = h1.astype(jnp.bfloat16)
    fh2_ref[...] = _mha_residual(
        fh1_ref[...], h1,
        Wq_ref[...], bq_ref[...], Wk_ref[...], bk_ref[...],
        Wv_ref[...], bv_ref[...], Wo_ref[...], bo_ref[...])


def _p3_body(xeb_ref, g_ref, h1b_ref, fh2_ref, f_ref, W1_ref, b1_ref,
             Wl_ref, bl_ref, out_ref):
    xe = xeb_ref[...].astype(jnp.float32)
    h1 = h1b_ref[...].astype(jnp.float32)
    m = g_ref[...] - _pairswap(h1)
    h2 = jnp.maximum(xe + _dot(m, W1_ref[...]) + b1_ref[...], 0.0)
    mail_sum = _segsum_k(h2)
    Wl = Wl_ref[...]
    out_ref[...] = (_dot(mail_sum, Wl[0:D]) + _dot(fh2_ref[...], Wl[D:2 * D])
                    + _dot(f_ref[...], Wl[2 * D:3 * D]) + bl_ref[...])


def _edge_spec(off):
    return pl.BlockSpec((BE, D), lambda i, o=off: (i + o, 0))


def _node_spec(off):
    return pl.BlockSpec((BN, D), lambda i, o=off: (i + o, 0))


def _w_spec(rows):
    return pl.BlockSpec((rows, D), lambda i: (0, 0))


def _b_spec():
    return pl.BlockSpec((1, D), lambda i: (0, 0))


def _make_sc_gather(rows_total):
    info = plsc.get_sparse_core_info()
    nw = info.num_cores * info.num_subcores
    per_w = rows_total // nw
    ch = 200
    n_ch = per_w // ch
    pairs = n_ch // 2
    tail = n_ch - 2 * pairs
    mesh = plsc.VectorSubcoreMesh(core_axis_name="c", subcore_axis_name="s")

    @functools.partial(
        pl.kernel,
        out_type=jax.ShapeDtypeStruct((rows_total, D), jnp.float32),
        mesh=mesh,
        scratch_types=[
            pltpu.VMEM((ch,), jnp.int32),
            pltpu.VMEM((ch,), jnp.int32),
            pltpu.VMEM((ch, D), jnp.float32),
            pltpu.VMEM((ch, D), jnp.float32),
            pltpu.SemaphoreType.DMA,
            pltpu.SemaphoreType.DMA,
            pltpu.SemaphoreType.DMA,
            pltpu.SemaphoreType.DMA,
        ],
    )
    def gather(table_hbm, idx_hbm, out_hbm, idx_a, idx_b, rows_a, rows_b,
               gs_a, gs_b, ss_a, ss_b):
        wid = lax.axis_index("s") * info.num_cores + lax.axis_index("c")
        base = wid * per_w
        idx_v = (idx_a, idx_b)
        rows_v = (rows_a, rows_b)
        gs = (gs_a, gs_b)
        ss = (ss_a, ss_b)

        def store_wait(b):
            pltpu.make_async_copy(rows_v[b], out_hbm.at[pl.ds(base, ch)],
                                  ss[b]).wait()

        def body(i, _):
            @pl.when(i > 0)
            def _():
                for b in range(2):
                    store_wait(b)
            handles = []
            for b in range(2):
                off = base + (2 * i + b) * ch
                pltpu.sync_copy(idx_hbm.at[pl.ds(off, ch)], idx_v[b])
                handles.append(
                    pltpu.async_copy(table_hbm.at[idx_v[b]], rows_v[b],
                                     gs[b]))
            for b in range(2):
                off = base + (2 * i + b) * ch
                handles[b].wait()
                pltpu.async_copy(rows_v[b], out_hbm.at[pl.ds(off, ch)],
                                 ss[b])
            return ()

        lax.fori_loop(0, pairs, body, ())
        for b in range(2):
            store_wait(b)
        if tail:
            off = base + 2 * pairs * ch
            pltpu.sync_copy(idx_hbm.at[pl.ds(off, ch)], idx_a)
            pltpu.async_copy(table_hbm.at[idx_a], rows_a, gs_a).wait()
            pltpu.sync_copy(rows_a, out_hbm.at[pl.ds(off, ch)])

    return gather


def kernel(f, x_e, src, Wq, bq, Wk, bk, Wv, bv, Wo, bo, W0, b0, W1, b1,
           Wl, bl):
    bq2, bk2, bv2, bo2, b02, b12, bl2 = (
        b.reshape(1, D) for b in (bq, bk, bv, bo, b0, b1, bl))

    p1bn = 400
    p1 = pl.pallas_call(
        _p1_body,
        grid=(N // p1bn,),
        in_specs=[pl.BlockSpec((p1bn * K, D), lambda i: (i, 0)),
                  pl.BlockSpec((p1bn, D), lambda i: (i, 0)),
                  _w_spec(D), _b_spec(), _w_spec(D), _b_spec(),
                  _w_spec(D), _b_spec(), _w_spec(D), _b_spec()],
        out_specs=[pl.BlockSpec((p1bn, D), lambda i: (i, 0)),
                   pl.BlockSpec((p1bn * K, D), lambda i: (i, 0))],
        out_shape=[jax.ShapeDtypeStruct((N, D), jnp.float32),
                   jax.ShapeDtypeStruct((E, D), jnp.bfloat16)],
    )
    fh1, xeb = p1(x_e, f, Wq, bq2, Wk, bk2, Wv, bv2, Wo, bo2)

    gathers = {u: _make_sc_gather(u * BE) for u in set(SLICES)}
    src_s = [src[OFFS[s] * BE:(OFFS[s] + SLICES[s]) * BE] for s in range(S)]

    def xeb_spec(off):
        return pl.BlockSpec((BE, D), lambda i, o=off: (i + o, 0))

    def p2_slice(s, g):
        off = OFFS[s]
        u = SLICES[s]
        call = pl.pallas_call(
            _p2_body,
            grid=(u,),
            in_specs=[xeb_spec(off),
                      pl.BlockSpec((BE, D), lambda i: (i, 0)),
                      _node_spec(off),
                      _w_spec(D), _b_spec(), _w_spec(D), _b_spec(),
                      _w_spec(D), _b_spec(), _w_spec(D), _b_spec(),
                      _w_spec(D), _b_spec()],
            out_specs=[pl.BlockSpec((BE, D), lambda i: (i, 0)),
                       pl.BlockSpec((BN, D), lambda i: (i, 0))],
            out_shape=[jax.ShapeDtypeStruct((u * BE, D), jnp.bfloat16),
                       jax.ShapeDtypeStruct((u * BN, D), jnp.float32)],
        )
        return call(xeb, g, fh1, Wq, bq2, Wk, bk2, Wv, bv2, Wo, bo2,
                    W0, b02)

    def p3_slice(s, g, h1b, fh2s):
        off = OFFS[s]
        u = SLICES[s]
        call = pl.pallas_call(
            _p3_body,
            grid=(u,),
            in_specs=[xeb_spec(off),
                      pl.BlockSpec((BE, D), lambda i: (i, 0)),
                      pl.BlockSpec((BE, D), lambda i: (i, 0)),
                      pl.BlockSpec((BN, D), lambda i: (i, 0)),
                      _node_spec(off), _w_spec(D), _b_spec(),
                      pl.BlockSpec((3 * D, D), lambda i: (0, 0)), _b_spec()],
            out_specs=pl.BlockSpec((BN, D), lambda i: (i, 0)),
            out_shape=jax.ShapeDtypeStruct((u * BN, D), jnp.float32),
        )
        return call(xeb, g, h1b, fh2s, f, W1, b12, Wl, bl2)

    g0 = [gathers[SLICES[s]](fh1, src_s[s]) for s in range(S)]
    p2_out = [p2_slice(s, g0[s]) for s in range(S)]
    h1b = [o[0] for o in p2_out]
    fh2s = [o[1] for o in p2_out]
    fh2 = jnp.concatenate(fh2s, axis=0)

    g1 = [gathers[SLICES[s]](fh2, src_s[s]) for s in range(S)]
    outs = [p3_slice(s, g1[s], h1b[s], fh2s[s]) for s in range(S)]
    return jnp.concatenate(outs, axis=0)
